# baseline (speedup 1.0000x reference)
import functools

import jax
import jax.numpy as jnp
from jax import lax
from jax.experimental import pallas as pl
from jax.experimental.pallas import tpu as pltpu
from jax.experimental.pallas import tpu_sc as plsc

B = 1024
D = 32
V = 100000
VBLK = 2000
NB = V // VBLK

_NC = 2
_NS = 16
_NW = _NC * _NS
_B_PER_W = B // _NW

@functools.cache
def _make_sc_gather():
    mesh = plsc.VectorSubcoreMesh(core_axis_name="c", subcore_axis_name="s")

    @functools.partial(
        pl.kernel,
        mesh=mesh,
        out_type=[
            jax.ShapeDtypeStruct((B, D), jnp.float32),
            jax.ShapeDtypeStruct((B,), jnp.float32),
        ],
        scratch_types=[
            pltpu.VMEM((_B_PER_W,), jnp.int32),
            pltpu.VMEM((_B_PER_W, D), jnp.float32),
            pltpu.VMEM((_B_PER_W,), jnp.float32),
            pltpu.SemaphoreType.DMA,
            pltpu.SemaphoreType.DMA,
        ],
        compiler_params=pltpu.CompilerParams(use_tc_tiling_on_sc=False),
    )
    def _sc_gather(emb_hbm, bias_hbm, idx_hbm, rows_out, bvals_out,
                   idx_v, rows_v, bvals_v, sem_r, sem_b):
        wid = lax.axis_index("s") * _NC + lax.axis_index("c")
        base = wid * _B_PER_W
        pltpu.sync_copy(idx_hbm.at[pl.ds(base, _B_PER_W)], idx_v)
        c_rows = pltpu.async_copy(emb_hbm.at[idx_v], rows_v, sem_r)
        c_bias = pltpu.async_copy(bias_hbm.at[idx_v], bvals_v, sem_b)
        c_rows.wait()
        c_bias.wait()
        pltpu.sync_copy(rows_v, rows_out.at[pl.ds(base, _B_PER_W)])
        pltpu.sync_copy(bvals_v, bvals_out.at[pl.ds(base, _B_PER_W)])

    return _sc_gather


_LOG2E = 1.4426950408889634
_LN2 = 0.6931471805599453


def _tc_body(movie_ref, emb_ref, bias_ref, gath_ref, biast_ref, out_ref,
             m_ref, s_ref):
    i = pl.program_id(0)
    movie = movie_ref[...]
    blk = lax.dot_general(
        movie * _LOG2E, emb_ref[...], (((1,), (1,)), ((), ())),
        preferred_element_type=jnp.float32)
    blk = blk + bias_ref[0] * _LOG2E
    bm = jnp.max(blk, axis=1, keepdims=True)

    @pl.when(i == 0)
    def _():
        m_ref[...] = jnp.full((B, 1), -1e30, jnp.float32)
        s_ref[...] = jnp.zeros((B, 1), jnp.float32)

    m_old = m_ref[...]
    m_new = jnp.maximum(m_old, bm)
    s_ref[...] = s_ref[...] * jnp.exp2(m_old - m_new) + jnp.sum(
        jnp.exp2(blk - m_new), axis=1, keepdims=True)
    m_ref[...] = m_new

    @pl.when(i == NB - 1)
    def _():
        tl = jnp.sum(movie * gath_ref[...], axis=1, keepdims=True)
        out_ref[...] = (m_ref[...] + jnp.log2(s_ref[...])) * _LN2 \
            - tl - biast_ref[...]


_tc_call = pl.pallas_call(
    _tc_body,
    grid=(NB,),
    in_specs=[
        pl.BlockSpec((B, D), lambda i: (0, 0)),
        pl.BlockSpec((VBLK, D), lambda i: (i, 0)),
        pl.BlockSpec((1, 1, VBLK), lambda i: (i, 0, 0)),
        pl.BlockSpec((B, D), lambda i: (0, 0)),
        pl.BlockSpec((B, 1), lambda i: (0, 0)),
    ],
    out_specs=pl.BlockSpec((B, 1), lambda i: (0, 0)),
    out_shape=jax.ShapeDtypeStruct((B, 1), jnp.float32),
    scratch_shapes=[
        pltpu.VMEM((B, 1), jnp.float32),
        pltpu.VMEM((B, 1), jnp.float32),
    ],
)


def kernel(movie_id_tensor, target_movie_ids, embedding, bias):
    idx = target_movie_ids.astype(jnp.int32)
    rows, bvals = _make_sc_gather()(embedding, bias, idx)
    loss = _tc_call(
        movie_id_tensor,
        embedding,
        bias.reshape(NB, 1, VBLK),
        rows,
        bvals.reshape(B, 1),
    )
    return loss.reshape(B)


# speedup vs baseline: 1.0570x; 1.0570x over previous
import functools

import jax
import jax.numpy as jnp
from jax import lax
from jax.experimental import pallas as pl
from jax.experimental.pallas import tpu as pltpu
from jax.experimental.pallas import tpu_sc as plsc

B = 1024
D = 32
V = 100000
VBLK = 5000
NB = V // VBLK

_NC = 2
_NS = 16
_NW = _NC * _NS
_B_PER_W = B // _NW

@functools.cache
def _make_sc_gather():
    mesh = plsc.VectorSubcoreMesh(core_axis_name="c", subcore_axis_name="s")

    @functools.partial(
        pl.kernel,
        mesh=mesh,
        out_type=[
            jax.ShapeDtypeStruct((B, D), jnp.float32),
            jax.ShapeDtypeStruct((B,), jnp.float32),
        ],
        scratch_types=[
            pltpu.VMEM((_B_PER_W,), jnp.int32),
            pltpu.VMEM((_B_PER_W, D), jnp.float32),
            pltpu.VMEM((_B_PER_W,), jnp.float32),
            pltpu.SemaphoreType.DMA,
            pltpu.SemaphoreType.DMA,
        ],
        compiler_params=pltpu.CompilerParams(use_tc_tiling_on_sc=False),
    )
    def _sc_gather(emb_hbm, bias_hbm, idx_hbm, rows_out, bvals_out,
                   idx_v, rows_v, bvals_v, sem_r, sem_b):
        wid = lax.axis_index("s") * _NC + lax.axis_index("c")
        base = wid * _B_PER_W
        pltpu.sync_copy(idx_hbm.at[pl.ds(base, _B_PER_W)], idx_v)
        c_rows = pltpu.async_copy(emb_hbm.at[idx_v], rows_v, sem_r)
        c_bias = pltpu.async_copy(bias_hbm.at[idx_v], bvals_v, sem_b)
        c_rows.wait()
        c_bias.wait()
        pltpu.sync_copy(rows_v, rows_out.at[pl.ds(base, _B_PER_W)])
        pltpu.sync_copy(bvals_v, bvals_out.at[pl.ds(base, _B_PER_W)])

    return _sc_gather


_LOG2E = 1.4426950408889634
_LN2 = 0.6931471805599453


def _tc_body(movie_ref, emb_ref, bias_ref, gath_ref, biast_ref, out_ref,
             m_ref, s_ref):
    i = pl.program_id(0)
    movie = movie_ref[...]
    blk = lax.dot_general(
        movie * _LOG2E, emb_ref[...], (((1,), (1,)), ((), ())),
        preferred_element_type=jnp.float32)
    blk = blk + bias_ref[0] * _LOG2E
    bm = jnp.max(blk, axis=1, keepdims=True)

    @pl.when(i == 0)
    def _():
        m_ref[...] = jnp.full((B, 1), -1e30, jnp.float32)
        s_ref[...] = jnp.zeros((B, 1), jnp.float32)

    m_old = m_ref[...]
    m_new = jnp.maximum(m_old, bm)
    s_ref[...] = s_ref[...] * jnp.exp2(m_old - m_new) + jnp.sum(
        jnp.exp2(blk - m_new), axis=1, keepdims=True)
    m_ref[...] = m_new

    @pl.when(i == NB - 1)
    def _():
        tl = jnp.sum(movie * gath_ref[...], axis=1, keepdims=True)
        out_ref[...] = (m_ref[...] + jnp.log2(s_ref[...])) * _LN2 \
            - tl - biast_ref[...]


_tc_call = pl.pallas_call(
    _tc_body,
    grid=(NB,),
    in_specs=[
        pl.BlockSpec((B, D), lambda i: (0, 0)),
        pl.BlockSpec((VBLK, D), lambda i: (i, 0)),
        pl.BlockSpec((1, 1, VBLK), lambda i: (i, 0, 0)),
        pl.BlockSpec((B, D), lambda i: (0, 0)),
        pl.BlockSpec((B, 1), lambda i: (0, 0)),
    ],
    out_specs=pl.BlockSpec((B, 1), lambda i: (0, 0)),
    out_shape=jax.ShapeDtypeStruct((B, 1), jnp.float32),
    scratch_shapes=[
        pltpu.VMEM((B, 1), jnp.float32),
        pltpu.VMEM((B, 1), jnp.float32),
    ],
)


def kernel(movie_id_tensor, target_movie_ids, embedding, bias):
    idx = target_movie_ids.astype(jnp.int32)
    rows, bvals = _make_sc_gather()(embedding, bias, idx)
    loss = _tc_call(
        movie_id_tensor,
        embedding,
        bias.reshape(NB, 1, VBLK),
        rows,
        bvals.reshape(B, 1),
    )
    return loss.reshape(B)


# speedup vs baseline: 1.1241x; 1.0635x over previous
import functools

import jax
import jax.numpy as jnp
from jax import lax
from jax.experimental import pallas as pl
from jax.experimental.pallas import tpu as pltpu
from jax.experimental.pallas import tpu_sc as plsc

B = 1024
D = 32
V = 100000
VBLK = 10000
NB = V // VBLK

_NC = 2
_NS = 16
_NW = _NC * _NS
_B_PER_W = B // _NW

@functools.cache
def _make_sc_gather():
    mesh = plsc.VectorSubcoreMesh(core_axis_name="c", subcore_axis_name="s")

    @functools.partial(
        pl.kernel,
        mesh=mesh,
        out_type=[
            jax.ShapeDtypeStruct((B, D), jnp.float32),
            jax.ShapeDtypeStruct((B,), jnp.float32),
        ],
        scratch_types=[
            pltpu.VMEM((_B_PER_W,), jnp.int32),
            pltpu.VMEM((_B_PER_W, D), jnp.float32),
            pltpu.VMEM((_B_PER_W,), jnp.float32),
            pltpu.SemaphoreType.DMA,
            pltpu.SemaphoreType.DMA,
        ],
        compiler_params=pltpu.CompilerParams(use_tc_tiling_on_sc=False),
    )
    def _sc_gather(emb_hbm, bias_hbm, idx_hbm, rows_out, bvals_out,
                   idx_v, rows_v, bvals_v, sem_r, sem_b):
        wid = lax.axis_index("s") * _NC + lax.axis_index("c")
        base = wid * _B_PER_W
        pltpu.sync_copy(idx_hbm.at[pl.ds(base, _B_PER_W)], idx_v)
        c_rows = pltpu.async_copy(emb_hbm.at[idx_v], rows_v, sem_r)
        c_bias = pltpu.async_copy(bias_hbm.at[idx_v], bvals_v, sem_b)
        c_rows.wait()
        c_bias.wait()
        pltpu.sync_copy(rows_v, rows_out.at[pl.ds(base, _B_PER_W)])
        pltpu.sync_copy(bvals_v, bvals_out.at[pl.ds(base, _B_PER_W)])

    return _sc_gather


_LOG2E = 1.4426950408889634
_LN2 = 0.6931471805599453


def _tc_body(movie_ref, emb_ref, bias_ref, gath_ref, biast_ref, out_ref,
             m_ref, s_ref):
    i = pl.program_id(0)
    movie = movie_ref[...]
    blk = lax.dot_general(
        movie * _LOG2E, emb_ref[...], (((1,), (1,)), ((), ())),
        preferred_element_type=jnp.float32)
    blk = blk + bias_ref[0] * _LOG2E
    bm = jnp.max(blk, axis=1, keepdims=True)

    @pl.when(i == 0)
    def _():
        m_ref[...] = jnp.full((B, 1), -1e30, jnp.float32)
        s_ref[...] = jnp.zeros((B, 1), jnp.float32)

    m_old = m_ref[...]
    m_new = jnp.maximum(m_old, bm)
    s_ref[...] = s_ref[...] * jnp.exp2(m_old - m_new) + jnp.sum(
        jnp.exp2(blk - m_new), axis=1, keepdims=True)
    m_ref[...] = m_new

    @pl.when(i == NB - 1)
    def _():
        tl = jnp.sum(movie * gath_ref[...], axis=1, keepdims=True)
        out_ref[...] = (m_ref[...] + jnp.log2(s_ref[...])) * _LN2 \
            - tl - biast_ref[...]


_tc_call = pl.pallas_call(
    _tc_body,
    grid=(NB,),
    in_specs=[
        pl.BlockSpec((B, D), lambda i: (0, 0)),
        pl.BlockSpec((VBLK, D), lambda i: (i, 0)),
        pl.BlockSpec((1, 1, VBLK), lambda i: (i, 0, 0)),
        pl.BlockSpec((B, D), lambda i: (0, 0)),
        pl.BlockSpec((B, 1), lambda i: (0, 0)),
    ],
    out_specs=pl.BlockSpec((B, 1), lambda i: (0, 0)),
    out_shape=jax.ShapeDtypeStruct((B, 1), jnp.float32),
    scratch_shapes=[
        pltpu.VMEM((B, 1), jnp.float32),
        pltpu.VMEM((B, 1), jnp.float32),
    ],
)


def kernel(movie_id_tensor, target_movie_ids, embedding, bias):
    idx = target_movie_ids.astype(jnp.int32)
    rows, bvals = _make_sc_gather()(embedding, bias, idx)
    loss = _tc_call(
        movie_id_tensor,
        embedding,
        bias.reshape(NB, 1, VBLK),
        rows,
        bvals.reshape(B, 1),
    )
    return loss.reshape(B)
